# baseline (speedup 1.0000x reference)
import jax
from jax.experimental import pallas as pl


def _add_pos_block(x_ref, pos_ref, o_ref):
    o_ref[...] = x_ref[...] + pos_ref[...][None, :, :]


def kernel(x, pos_table):
    batch, seq_len, dim = x.shape
    block_s = 512
    grid = (seq_len // block_s,)
    return pl.pallas_call(
        _add_pos_block,
        grid=grid,
        in_specs=[
            pl.BlockSpec((batch, block_s, dim), lambda i: (0, i, 0)),
            pl.BlockSpec((block_s, dim), lambda i: (i, 0)),
        ],
        out_specs=pl.BlockSpec((batch, block_s, dim), lambda i: (0, i, 0)),
        out_shape=jax.ShapeDtypeStruct((batch, seq_len, dim), x.dtype),
    )(x, pos_table[:seq_len])
